# Initial kernel scaffold; baseline (speedup 1.0000x reference)
import jax
import jax.numpy as jnp
from jax.experimental import pallas as pl


def kernel(hidden_states, Wg, W1, W2, W3):
    raise NotImplementedError("write your pallas kernel here")


# speedup vs baseline: 1.4660x; 1.4660x over previous
import functools

import jax
import jax.numpy as jnp
from jax.experimental import pallas as pl
from jax.experimental.pallas import tpu as pltpu


def _moe_dense_body(x_ref, wg_ref, w1_ref, w2_ref, w3_ref, out_ref, wfull_ref):
    e = pl.program_id(0)
    fc = pl.program_id(1)
    num_e = pl.num_programs(0)

    @pl.when((e == 0) & (fc == 0))
    def _router():
        x = x_ref[...]
        logits = jax.lax.dot_general(
            x, wg_ref[...], (((1,), (1,)), ((), ())),
            preferred_element_type=jnp.float32)
        lane = jax.lax.broadcasted_iota(jnp.int32, logits.shape, 1)
        m1 = jnp.max(logits, axis=1, keepdims=True)
        i1 = jnp.min(jnp.where(logits == m1, lane, num_e), axis=1, keepdims=True)
        masked = jnp.where(lane == i1, -jnp.inf, logits)
        m2 = jnp.max(masked, axis=1, keepdims=True)
        i2 = jnp.min(jnp.where(masked == m2, lane, num_e), axis=1, keepdims=True)
        z = jnp.exp(m2 - m1)
        p1 = 1.0 / (1.0 + z)
        p2 = z / (1.0 + z)
        wfull_ref[...] = (jnp.where(lane == i1, p1, 0.0)
                          + jnp.where(lane == i2, p2, 0.0))
        out_ref[...] = jnp.zeros_like(out_ref)

    x = x_ref[...]
    lane = jax.lax.broadcasted_iota(jnp.int32, wfull_ref.shape, 1)
    w_col = jnp.sum(wfull_ref[...] * (lane == e), axis=1, keepdims=True)
    a = jax.lax.dot_general(x, w1_ref[0], (((1,), (1,)), ((), ())),
                            preferred_element_type=jnp.float32)
    b = jax.lax.dot_general(x, w3_ref[0], (((1,), (1,)), ((), ())),
                            preferred_element_type=jnp.float32)
    hmid = (a * jax.lax.logistic(a)) * b
    contrib = jax.lax.dot_general(hmid, w2_ref[0], (((1,), (1,)), ((), ())),
                                  preferred_element_type=jnp.float32)
    out_ref[...] += w_col * contrib


def kernel(hidden_states, Wg, W1, W2, W3):
    b, s, h = hidden_states.shape
    e, ffn, _ = W1.shape
    x = hidden_states.reshape(s, h)
    fc = min(1024, ffn)
    nfc = ffn // fc

    out = pl.pallas_call(
        _moe_dense_body,
        grid=(e, nfc),
        in_specs=[
            pl.BlockSpec((s, h), lambda ei, ci: (0, 0)),
            pl.BlockSpec((e, h), lambda ei, ci: (0, 0)),
            pl.BlockSpec((1, fc, h), lambda ei, ci: (ei, ci, 0)),
            pl.BlockSpec((1, h, fc), lambda ei, ci: (ei, 0, ci)),
            pl.BlockSpec((1, fc, h), lambda ei, ci: (ei, ci, 0)),
        ],
        out_specs=pl.BlockSpec((s, h), lambda ei, ci: (0, 0)),
        out_shape=jax.ShapeDtypeStruct((s, h), jnp.float32),
        scratch_shapes=[pltpu.VMEM((s, e), jnp.float32)],
    )(x, Wg, W1, W2, W3)
    return out.reshape(b, s, h)


# speedup vs baseline: 1.5776x; 1.0762x over previous
import functools

import jax
import jax.numpy as jnp
from jax import lax
from jax.experimental import pallas as pl
from jax.experimental.pallas import tpu as pltpu
from jax.experimental.pallas import tpu_sc as plsc

BLK = 256
FC = 512
TOP_K = 2


def _cumsum_rows(m):
    t = m.shape[0]
    s = 1
    while s < t:
        shifted = lax.pad(lax.slice(m, (0, 0), (t - s, m.shape[1])),
                          jnp.float32(0.0), ((s, 0, 0), (0, 0, 0)))
        m = m + shifted
        s *= 2
    return m


def _router_body(x_ref, wg_ref, p0_ref, p1_ref, w0_ref, w1_ref, te_ref):
    x = x_ref[...]
    num_e = wg_ref.shape[0]
    logits = lax.dot_general(x, wg_ref[...], (((1,), (1,)), ((), ())),
                             preferred_element_type=jnp.float32)
    lane = lax.broadcasted_iota(jnp.int32, logits.shape, 1)
    m1 = jnp.max(logits, axis=1, keepdims=True)
    i1 = jnp.min(jnp.where(logits == m1, lane, num_e), axis=1, keepdims=True)
    masked = jnp.where(lane == i1, -jnp.inf, logits)
    m2 = jnp.max(masked, axis=1, keepdims=True)
    i2 = jnp.min(jnp.where(masked == m2, lane, num_e), axis=1, keepdims=True)
    z = jnp.exp(m2 - m1)
    w0_ref[...] = jnp.broadcast_to(1.0 / (1.0 + z), w0_ref.shape)
    w1_ref[...] = jnp.broadcast_to(z / (1.0 + z), w1_ref.shape)

    oh1 = (lane == i1).astype(jnp.float32)
    oh2 = (lane == i2).astype(jnp.float32)
    ohc = jnp.concatenate([oh1, oh2], axis=1)
    csum = _cumsum_rows(ohc)
    t_dim = x.shape[0]
    totals = lax.slice(csum, (t_dim - 1, 0), (t_dim, 2 * num_e))
    count1 = lax.slice(totals, (0, 0), (1, num_e))
    count2 = lax.slice(totals, (0, num_e), (1, 2 * num_e))
    counts = (count1 + count2).astype(jnp.int32)
    cum1 = lax.slice(csum, (0, 0), (t_dim, num_e))
    cum2 = lax.slice(csum, (0, num_e), (t_dim, 2 * num_e))

    nt_e = (counts + (BLK - 1)) // BLK
    incl = nt_e
    s = 1
    while s < num_e:
        shifted = lax.pad(lax.slice(incl, (0, 0), (1, num_e - s)),
                          jnp.int32(0), ((0, 0, 0), (s, 0, 0)))
        incl = incl + shifted
        s *= 2
    tile_start = incl - nt_e
    row_off = (tile_start * BLK).astype(jnp.float32)

    r0 = jnp.sum((cum1 - oh1 + row_off) * oh1, axis=1, keepdims=True)
    r1 = jnp.sum((count1 + cum2 - oh2 + row_off) * oh2, axis=1, keepdims=True)
    p0_ref[...] = r0.astype(jnp.int32)
    p1_ref[...] = r1.astype(jnp.int32)

    jiota = lax.broadcasted_iota(jnp.int32, (1, te_ref.shape[1]), 1)
    acc = jnp.zeros((1, te_ref.shape[1]), jnp.int32)
    for e in range(num_e):
        ts_e = lax.slice(tile_start, (0, e), (1, e + 1))
        acc = acc + jnp.where(ts_e <= jiota, 1, 0)
    te_ref[...] = jnp.broadcast_to(acc - 1, te_ref.shape)


def _grouped_ffn_body(te_ref, xs_ref, w1_ref, w2_ref, w3_ref, out_ref, acc_ref):
    fc = pl.program_id(0)
    t = pl.program_id(1)
    nfc = pl.num_programs(0)
    xs = xs_ref[...]
    a = lax.dot_general(xs, w1_ref[0], (((1,), (1,)), ((), ())),
                        preferred_element_type=jnp.float32)
    b = lax.dot_general(xs, w3_ref[0], (((1,), (1,)), ((), ())),
                        preferred_element_type=jnp.float32)
    hmid = (a * lax.logistic(a)) * b
    contrib = lax.dot_general(hmid, w2_ref[0], (((1,), (1,)), ((), ())),
                              preferred_element_type=jnp.float32)

    @pl.when(fc == 0)
    def _():
        acc_ref[pl.ds(t * BLK, BLK), :] = contrib

    @pl.when(fc != 0)
    def _():
        acc_ref[pl.ds(t * BLK, BLK), :] += contrib

    @pl.when(fc == nfc - 1)
    def _():
        out_ref[...] = acc_ref[pl.ds(t * BLK, BLK), :]


def _make_sc_scatter(t_dim, h, stotal, nw):
    chunk = t_dim // nw
    mesh = plsc.VectorSubcoreMesh(core_axis_name="c", subcore_axis_name="s")

    @functools.partial(
        pl.kernel, mesh=mesh,
        out_type=jax.ShapeDtypeStruct((stotal, h), jnp.float32),
        scratch_types=[
            pltpu.VMEM((chunk,), jnp.int32),
            pltpu.VMEM((chunk,), jnp.int32),
            pltpu.VMEM((chunk, h), jnp.float32),
            pltpu.SemaphoreType.DMA,
        ],
    )
    def sc_scatter(x_hbm, p0_hbm, p1_hbm, xs_hbm, idx0_v, idx1_v, rows_v, sem):
        wid = lax.axis_index("s") * 2 + lax.axis_index("c")
        base = wid * chunk
        pltpu.sync_copy(p0_hbm.at[pl.ds(base, chunk)], idx0_v)
        pltpu.sync_copy(p1_hbm.at[pl.ds(base, chunk)], idx1_v)
        pltpu.sync_copy(x_hbm.at[pl.ds(base, chunk)], rows_v)
        pltpu.async_copy(rows_v, xs_hbm.at[idx0_v], sem).wait()
        pltpu.async_copy(rows_v, xs_hbm.at[idx1_v], sem).wait()

    return sc_scatter


def _make_sc_combine(t_dim, h, stotal, nw):
    chunk = t_dim // nw
    sub = min(chunk, 32)
    nsub = chunk // sub
    nlane = h // 16
    mesh = plsc.VectorSubcoreMesh(core_axis_name="c", subcore_axis_name="s")

    @functools.partial(
        pl.kernel, mesh=mesh,
        out_type=jax.ShapeDtypeStruct((t_dim, h), jnp.float32),
        scratch_types=[
            pltpu.VMEM((sub,), jnp.int32),
            pltpu.VMEM((sub,), jnp.int32),
            pltpu.VMEM((sub, 16), jnp.float32),
            pltpu.VMEM((sub, 16), jnp.float32),
            pltpu.VMEM((sub, h), jnp.float32),
            pltpu.VMEM((sub, h), jnp.float32),
            pltpu.VMEM((sub, h), jnp.float32),
            pltpu.SemaphoreType.DMA,
        ],
    )
    def sc_combine(rows_hbm, p0_hbm, p1_hbm, w0_hbm, w1_hbm, out_hbm,
                   idx0_v, idx1_v, w0_v, w1_v, buf0, buf1, accv, sem):
        wid = lax.axis_index("s") * 2 + lax.axis_index("c")
        for half in range(nsub):
            base = wid * chunk + half * sub
            pltpu.sync_copy(p0_hbm.at[pl.ds(base, sub)], idx0_v)
            pltpu.sync_copy(p1_hbm.at[pl.ds(base, sub)], idx1_v)
            pltpu.sync_copy(w0_hbm.at[pl.ds(base, sub)], w0_v)
            pltpu.sync_copy(w1_hbm.at[pl.ds(base, sub)], w1_v)
            pltpu.async_copy(rows_hbm.at[idx0_v], buf0, sem).wait()
            pltpu.async_copy(rows_hbm.at[idx1_v], buf1, sem).wait()

            def tok_body(i, _):
                a = w0_v[i, :]
                b = w1_v[i, :]

                def lane_body(j, _):
                    accv[i, pl.ds(j * 16, 16)] = (
                        buf0[i, pl.ds(j * 16, 16)] * a
                        + buf1[i, pl.ds(j * 16, 16)] * b)
                    return 0

                lax.fori_loop(0, nlane, lane_body, 0, unroll=4)
                return 0

            lax.fori_loop(0, sub, tok_body, 0)
            pltpu.sync_copy(accv, out_hbm.at[pl.ds(base, sub)])

    return sc_combine


def kernel(hidden_states, Wg, W1, W2, W3):
    bsz, s, h = hidden_states.shape
    e, ffn, _ = W1.shape
    t_dim = bsz * s
    x = hidden_states.reshape(t_dim, h)
    nt = (t_dim * TOP_K) // BLK + e - 1
    stotal = nt * BLK
    nfc = ffn // FC
    te_lanes = 128

    p0c, p1c, w0c, w1c, te_mat = pl.pallas_call(
        _router_body,
        grid=(1,),
        in_specs=[
            pl.BlockSpec((t_dim, h), lambda i: (0, 0)),
            pl.BlockSpec((e, h), lambda i: (0, 0)),
        ],
        out_specs=[
            pl.BlockSpec((t_dim, 1), lambda i: (0, 0)),
            pl.BlockSpec((t_dim, 1), lambda i: (0, 0)),
            pl.BlockSpec((t_dim, 16), lambda i: (0, 0)),
            pl.BlockSpec((t_dim, 16), lambda i: (0, 0)),
            pl.BlockSpec((8, te_lanes), lambda i: (0, 0)),
        ],
        out_shape=[
            jax.ShapeDtypeStruct((t_dim, 1), jnp.int32),
            jax.ShapeDtypeStruct((t_dim, 1), jnp.int32),
            jax.ShapeDtypeStruct((t_dim, 16), jnp.float32),
            jax.ShapeDtypeStruct((t_dim, 16), jnp.float32),
            jax.ShapeDtypeStruct((8, te_lanes), jnp.int32),
        ],
    )(x, Wg)
    p0 = p0c.reshape(t_dim)
    p1 = p1c.reshape(t_dim)
    w0 = w0c
    w1 = w1c
    tile_expert = te_mat[0, :nt]

    xs = _make_sc_scatter(t_dim, h, stotal, 32)(x, p0, p1)

    rows = pl.pallas_call(
        _grouped_ffn_body,
        grid_spec=pltpu.PrefetchScalarGridSpec(
            num_scalar_prefetch=1,
            grid=(nfc, nt),
            in_specs=[
                pl.BlockSpec((BLK, h), lambda fc, t, te: (t, 0)),
                pl.BlockSpec((1, FC, h), lambda fc, t, te: (te[t], fc, 0)),
                pl.BlockSpec((1, h, FC), lambda fc, t, te: (te[t], 0, fc)),
                pl.BlockSpec((1, FC, h), lambda fc, t, te: (te[t], fc, 0)),
            ],
            out_specs=pl.BlockSpec(
                (BLK, h), lambda fc, t, te: (jnp.where(fc == nfc - 1, t, 0), 0)),
            scratch_shapes=[pltpu.VMEM((stotal, h), jnp.float32)],
        ),
        out_shape=jax.ShapeDtypeStruct((stotal, h), jnp.float32),
    )(tile_expert, xs, W1, W2, W3)

    final = _make_sc_combine(t_dim, h, stotal, 32)(rows, p0, p1, w0, w1)
    return final.reshape(bsz, s, h)
